# Initial kernel scaffold; baseline (speedup 1.0000x reference)
import jax
import jax.numpy as jnp
from jax.experimental import pallas as pl


def kernel(structure_data, semantic_emb, hyperedge_index, batch_indices, emb_table, W1, b1, W2, b2, protos, Wc, bc):
    raise NotImplementedError("write your pallas kernel here")


# speedup vs baseline: 2.4134x; 2.4134x over previous
import functools

import jax
import jax.numpy as jnp
from jax import lax
from jax.experimental import pallas as pl
from jax.experimental.pallas import tpu as pltpu
from jax.experimental.pallas import tpu_sc as plsc

N_DRUGS = 10000
N_HE = 10000
N_SEG = 10000
HID = 128
N_SE = 32
N_PROTO = 20
N_INC = 320000
BATCH = 4096

NC = 2
NS = 16
CH = 128
NCH = 2560
K = NCH // (NC * NS)
PAD_INC = NCH * CH
SEG_PAD = 10240
DUMMY = N_SEG
ZROWS = SEG_PAD // NS


def _sc_mesh():
    return plsc.VectorSubcoreMesh(core_axis_name="c", subcore_axis_name="s")


def _sc_segsum(table, gth_ch, sct_ch, zeros_hbm):
    @functools.partial(
        pl.kernel,
        mesh=_sc_mesh(),
        out_type=jax.ShapeDtypeStruct((NC, SEG_PAD, HID), jnp.float32),
        scratch_types=[
            pltpu.VMEM((K // 2, 1, CH), jnp.int32),
            pltpu.VMEM((K // 2, 1, CH), jnp.int32),
            pltpu.VMEM((CH, HID), jnp.float32),
            pltpu.VMEM((CH, HID), jnp.float32),
            pltpu.VMEM_SHARED((SEG_PAD, HID), jnp.float32),
            pltpu.SemaphoreType.DMA,
            pltpu.SemaphoreType.DMA,
        ],
    )
    def k(table_hbm, gidx_hbm, sidx_hbm, zero_hbm, out_hbm,
          gidx, sidx, buf0, buf1, acc, sem0, sem1):
        cid = lax.axis_index("c")
        sid = lax.axis_index("s")
        tbase = (cid * NS + sid) * K
        KH = K // 2
        pltpu.sync_copy(zero_hbm, acc.at[pl.ds(sid * ZROWS, ZROWS)])
        plsc.subcore_barrier()

        def body(i, _):
            j = 2 * i
            c0 = pltpu.async_copy(table_hbm.at[gidx.at[j, 0]], buf0, sem0)
            c1 = pltpu.async_copy(table_hbm.at[gidx.at[j + 1, 0]], buf1, sem1)
            c0.wait()
            pltpu.sync_copy(buf0, acc.at[sidx.at[j, 0]], add=True)
            c1.wait()
            pltpu.sync_copy(buf1, acc.at[sidx.at[j + 1, 0]], add=True)
            return 0

        for h in range(2):
            pltpu.sync_copy(gidx_hbm.at[pl.ds(tbase + h * KH, KH)], gidx)
            pltpu.sync_copy(sidx_hbm.at[pl.ds(tbase + h * KH, KH)], sidx)
            lax.fori_loop(0, KH // 2, body, 0)
        plsc.subcore_barrier()
        pltpu.sync_copy(
            acc.at[pl.ds(sid * ZROWS, ZROWS)],
            out_hbm.at[cid, pl.ds(sid * ZROWS, ZROWS)])

    return k(table, gth_ch, sct_ch, zeros_hbm)


HROWS = SEG_PAD // CH


def _sc_counts(src_ch, dst_ch, iota_hbm, zeros_hbm):
    @functools.partial(
        pl.kernel,
        mesh=_sc_mesh(),
        compiler_params=pltpu.CompilerParams(needs_layout_passes=False),
        out_type=jax.ShapeDtypeStruct((NC, 2, HROWS, CH), jnp.float32),
        scratch_types=[
            pltpu.VMEM((K, 1, CH), jnp.int32),
            pltpu.VMEM((K, 1, CH), jnp.int32),
            pltpu.VMEM((HROWS,), jnp.int32),
            pltpu.VMEM((HROWS, CH), jnp.float32),
            pltpu.VMEM((HROWS, CH), jnp.float32),
            pltpu.VMEM_SHARED((HROWS, CH), jnp.float32),
            pltpu.VMEM_SHARED((HROWS, CH), jnp.float32),
        ],
    )
    def k(sidx_hbm, didx_hbm, iota_h, zero_h, out_hbm,
          sidx, didx, iota_v, hist_s, hist_d, acc_s, acc_d):
        cid = lax.axis_index("c")
        sid = lax.axis_index("s")
        tbase = (cid * NS + sid) * K
        pltpu.sync_copy(sidx_hbm.at[pl.ds(tbase, K)], sidx)
        pltpu.sync_copy(didx_hbm.at[pl.ds(tbase, K)], didx)
        pltpu.sync_copy(iota_h, iota_v)
        pltpu.sync_copy(zero_h.at[pl.ds(0, HROWS)], hist_s)
        pltpu.sync_copy(zero_h.at[pl.ds(0, HROWS)], hist_d)

        @pl.when(sid == 0)
        def _():
            pltpu.sync_copy(zero_h.at[pl.ds(0, HROWS)], acc_s)
            pltpu.sync_copy(zero_h.at[pl.ds(0, HROWS)], acc_d)

        plsc.subcore_barrier()
        ones = jnp.ones((16,), jnp.float32)

        def chunk(j, _):
            def vec(l, _2):
                vs = sidx[j, 0, pl.ds(l * 16, 16)]
                vd = didx[j, 0, pl.ds(l * 16, 16)]
                plsc.addupdate_scatter(
                    hist_s,
                    [lax.shift_right_logical(vs, 7), lax.bitwise_and(vs, 127)],
                    ones)
                plsc.addupdate_scatter(
                    hist_d,
                    [lax.shift_right_logical(vd, 7), lax.bitwise_and(vd, 127)],
                    ones)
                return 0

            lax.fori_loop(0, CH // 16, vec, 0)
            return 0

        lax.fori_loop(0, K, chunk, 0)
        pltpu.sync_copy(hist_s, acc_s.at[iota_v], add=True)
        pltpu.sync_copy(hist_d, acc_d.at[iota_v], add=True)
        plsc.subcore_barrier()

        @pl.when(sid == 0)
        def _():
            pltpu.sync_copy(acc_s, out_hbm.at[cid, 0])
            pltpu.sync_copy(acc_d, out_hbm.at[cid, 1])

    return k(src_ch, dst_ch, iota_hbm, zeros_hbm)


def _sc_batch_gather(table, bi_ch):
    @functools.partial(
        pl.kernel,
        mesh=_sc_mesh(),
        out_type=jax.ShapeDtypeStruct((BATCH, HID), jnp.float32),
        scratch_types=[
            pltpu.VMEM((1, CH), jnp.int32),
            pltpu.VMEM((CH, HID), jnp.float32),
            pltpu.SemaphoreType.DMA,
        ],
    )
    def k(t_h, bi_h, out_h, idxv, rows, sem):
        cid = lax.axis_index("c")
        sid = lax.axis_index("s")
        wid = cid * NS + sid
        pltpu.sync_copy(bi_h.at[wid], idxv)
        pltpu.async_copy(t_h.at[idxv.at[0]], rows, sem).wait()
        pltpu.sync_copy(rows, out_h.at[pl.ds(wid * CH, CH)])

    return k(table, bi_ch)


def _tc_scale(acc2, cnt2):
    def body(a_ref, c_ref, o_ref):
        a = a_ref[0] + a_ref[1]
        c = c_ref[0] + c_ref[1]
        o_ref[...] = a / jnp.maximum(c, 1.0)

    return pl.pallas_call(
        body,
        out_shape=jax.ShapeDtypeStruct((SEG_PAD, HID), jnp.float32),
    )(acc2, cnt2)


def _tc_matmul(acc2, cnt2, W, b):
    def body(a_ref, c_ref, w_ref, b_ref, o_ref):
        a = a_ref[0] + a_ref[1]
        c = c_ref[0] + c_ref[1]
        x = a / jnp.maximum(c, 1.0)
        y = jnp.dot(x, w_ref[...], preferred_element_type=jnp.float32)
        o_ref[...] = jnp.maximum(y + b_ref[...][None, :], 0.0)

    return pl.pallas_call(
        body,
        out_shape=jax.ShapeDtypeStruct((SEG_PAD, HID), jnp.float32),
    )(acc2, cnt2, W, b)


def _tc_head(bt_rows, protos, Wc, bc):
    BB = 512
    NB = BATCH // BB

    def body(bt_ref, p_ref, wc_ref, bc_ref, o_ref):
        bt = bt_ref[...]
        p = p_ref[...]
        pf = p.reshape(N_SE * N_PROTO, HID)
        sT = lax.dot_general(
            pf, bt, (((1,), (1,)), ((), ())),
            preferred_element_type=jnp.float32)
        s3 = sT.reshape(N_SE, N_PROTO, BB)
        m = jnp.max(s3, axis=1, keepdims=True)
        e = jnp.exp(s3 - m)
        attn = e / jnp.sum(e, axis=1, keepdims=True)
        wc = wc_ref[...]
        q = jnp.sum(p * wc[:, None, :], axis=-1)
        contrib = jnp.sum(attn * q[:, :, None], axis=1)
        baseT = lax.dot_general(
            wc, bt, (((1,), (1,)), ((), ())),
            preferred_element_type=jnp.float32)
        logitsT = baseT + contrib + bc_ref[...][:, None]
        o_ref[...] = jnp.transpose(1.0 / (1.0 + jnp.exp(-logitsT)))

    return pl.pallas_call(
        body,
        grid=(NB,),
        in_specs=[
            pl.BlockSpec((BB, HID), lambda i: (i, 0)),
            pl.BlockSpec((N_SE, N_PROTO, HID), lambda i: (0, 0, 0)),
            pl.BlockSpec((N_SE, HID), lambda i: (0, 0)),
            pl.BlockSpec((N_SE,), lambda i: (0,)),
        ],
        out_specs=pl.BlockSpec((BB, N_SE), lambda i: (i, 0)),
        out_shape=jax.ShapeDtypeStruct((BATCH, N_SE), jnp.float32),
    )(bt_rows, protos, Wc, bc)


def kernel(structure_data, semantic_emb, hyperedge_index, batch_indices,
           emb_table, W1, b1, W2, b2, protos, Wc, bc):
    del structure_data, semantic_emb
    hi = hyperedge_index.astype(jnp.int32)
    src, dst = hi[0], hi[1]
    npad = PAD_INC - N_INC
    srcA = jnp.concatenate([src, jnp.zeros((npad,), jnp.int32)])
    dstA = jnp.concatenate([dst, jnp.full((npad,), DUMMY, jnp.int32)])
    dstB = jnp.concatenate([dst, jnp.zeros((npad,), jnp.int32)])
    srcB = jnp.concatenate([src, jnp.full((npad,), DUMMY, jnp.int32)])
    srcA = srcA.reshape(NCH, 1, CH)
    dstA = dstA.reshape(NCH, 1, CH)
    srcB = srcB.reshape(NCH, 1, CH)
    dstB = dstB.reshape(NCH, 1, CH)
    bi = batch_indices.astype(jnp.int32).reshape(NC * NS, 1, CH)

    zeros_hid = jnp.zeros((ZROWS, HID), jnp.float32)
    iota_h = jnp.arange(HROWS, dtype=jnp.int32)

    cnt = _sc_counts(srcB, dstA, iota_h, zeros_hid)
    cnt_nd = cnt[:, 0].reshape(NC, SEG_PAD, 1)
    cnt_he = cnt[:, 1].reshape(NC, SEG_PAD, 1)

    acc_he1 = _sc_segsum(emb_table, srcA, dstA, zeros_hid)
    he1 = _tc_scale(acc_he1, cnt_he)
    acc_nd1 = _sc_segsum(he1, dstB, srcB, zeros_hid)
    x2 = _tc_matmul(acc_nd1, cnt_nd, W1, b1)
    acc_he2 = _sc_segsum(x2, srcA, dstA, zeros_hid)
    he2 = _tc_scale(acc_he2, cnt_he)
    acc_nd2 = _sc_segsum(he2, dstB, srcB, zeros_hid)
    x3 = _tc_matmul(acc_nd2, cnt_nd, W2, b2)
    acc_t = _sc_segsum(x3, srcA, dstA, zeros_hid)
    treatment = _tc_scale(acc_t, cnt_he)

    bt_rows = _sc_batch_gather(treatment, bi)
    return _tc_head(bt_rows, protos, Wc, bc)


# speedup vs baseline: 2.5534x; 1.0580x over previous
import functools

import jax
import jax.numpy as jnp
from jax import lax
from jax.experimental import pallas as pl
from jax.experimental.pallas import tpu as pltpu
from jax.experimental.pallas import tpu_sc as plsc

N_DRUGS = 10000
N_HE = 10000
N_SEG = 10000
HID = 128
N_SE = 32
N_PROTO = 20
N_INC = 320000
BATCH = 4096

NC = 2
NS = 16
CH = 128
NCH = 2560
K = NCH // (NC * NS)
PAD_INC = NCH * CH
SEG_PAD = 10240
DUMMY = N_SEG
ZROWS = SEG_PAD // NS


def _sc_mesh():
    return plsc.VectorSubcoreMesh(core_axis_name="c", subcore_axis_name="s")


def _sc_segsum(table, gth_ch, sct_ch, zeros_hbm):
    @functools.partial(
        pl.kernel,
        mesh=_sc_mesh(),
        out_type=jax.ShapeDtypeStruct((NC, SEG_PAD, HID), jnp.float32),
        scratch_types=[
            pltpu.VMEM((K // 2, 1, CH), jnp.int32),
            pltpu.VMEM((K // 2, 1, CH), jnp.int32),
            pltpu.VMEM((CH, HID), jnp.float32),
            pltpu.VMEM((CH, HID), jnp.float32),
            pltpu.VMEM_SHARED((SEG_PAD, HID), jnp.float32),
            pltpu.SemaphoreType.DMA,
            pltpu.SemaphoreType.DMA,
            pltpu.SemaphoreType.DMA,
            pltpu.SemaphoreType.DMA,
        ],
    )
    def k(table_hbm, gidx_hbm, sidx_hbm, zero_hbm, out_hbm,
          gidx, sidx, buf0, buf1, acc, sem0, sem1, asem0, asem1):
        cid = lax.axis_index("c")
        sid = lax.axis_index("s")
        tbase = (cid * NS + sid) * K
        KH = K // 2
        pltpu.sync_copy(zero_hbm, acc.at[pl.ds(sid * ZROWS, ZROWS)])
        plsc.subcore_barrier()

        def gstart(j, buf, sem):
            pltpu.async_copy(table_hbm.at[gidx.at[j, 0]], buf, sem)

        def gwait(j, buf, sem):
            pltpu.make_async_copy(table_hbm.at[gidx.at[j, 0]], buf, sem).wait()

        for h in range(2):
            pltpu.sync_copy(gidx_hbm.at[pl.ds(tbase + h * KH, KH)], gidx)
            pltpu.sync_copy(sidx_hbm.at[pl.ds(tbase + h * KH, KH)], sidx)
            gstart(0, buf0, sem0)
            gstart(1, buf1, sem1)

            def body(i, _):
                j = 2 * i
                gwait(j, buf0, sem0)
                a0 = pltpu.async_copy(buf0, acc.at[sidx.at[j, 0]], asem0,
                                      add=True)
                gwait(j + 1, buf1, sem1)
                a1 = pltpu.async_copy(buf1, acc.at[sidx.at[j + 1, 0]], asem1,
                                      add=True)
                a0.wait()
                gstart(j + 2, buf0, sem0)
                a1.wait()
                gstart(j + 3, buf1, sem1)
                return 0

            lax.fori_loop(0, KH // 2 - 1, body, 0)
            j = KH - 2
            gwait(j, buf0, sem0)
            a0 = pltpu.async_copy(buf0, acc.at[sidx.at[j, 0]], asem0, add=True)
            gwait(j + 1, buf1, sem1)
            a1 = pltpu.async_copy(buf1, acc.at[sidx.at[j + 1, 0]], asem1,
                                  add=True)
            a0.wait()
            a1.wait()
        plsc.subcore_barrier()
        pltpu.sync_copy(
            acc.at[pl.ds(sid * ZROWS, ZROWS)],
            out_hbm.at[cid, pl.ds(sid * ZROWS, ZROWS)])

    return k(table, gth_ch, sct_ch, zeros_hbm)


HROWS = SEG_PAD // CH


def _sc_counts(src_ch, dst_ch, iota_hbm, zeros_hbm):
    @functools.partial(
        pl.kernel,
        mesh=_sc_mesh(),
        compiler_params=pltpu.CompilerParams(needs_layout_passes=False),
        out_type=jax.ShapeDtypeStruct((NC, 2, HROWS, CH), jnp.float32),
        scratch_types=[
            pltpu.VMEM((K, 1, CH), jnp.int32),
            pltpu.VMEM((K, 1, CH), jnp.int32),
            pltpu.VMEM((HROWS,), jnp.int32),
            pltpu.VMEM((HROWS, CH), jnp.float32),
            pltpu.VMEM((HROWS, CH), jnp.float32),
            pltpu.VMEM_SHARED((HROWS, CH), jnp.float32),
            pltpu.VMEM_SHARED((HROWS, CH), jnp.float32),
        ],
    )
    def k(sidx_hbm, didx_hbm, iota_h, zero_h, out_hbm,
          sidx, didx, iota_v, hist_s, hist_d, acc_s, acc_d):
        cid = lax.axis_index("c")
        sid = lax.axis_index("s")
        tbase = (cid * NS + sid) * K
        pltpu.sync_copy(sidx_hbm.at[pl.ds(tbase, K)], sidx)
        pltpu.sync_copy(didx_hbm.at[pl.ds(tbase, K)], didx)
        pltpu.sync_copy(iota_h, iota_v)
        pltpu.sync_copy(zero_h.at[pl.ds(0, HROWS)], hist_s)
        pltpu.sync_copy(zero_h.at[pl.ds(0, HROWS)], hist_d)

        @pl.when(sid == 0)
        def _():
            pltpu.sync_copy(zero_h.at[pl.ds(0, HROWS)], acc_s)
            pltpu.sync_copy(zero_h.at[pl.ds(0, HROWS)], acc_d)

        plsc.subcore_barrier()
        ones = jnp.ones((16,), jnp.float32)

        def chunk(j, _):
            def vec(l, _2):
                vs = sidx[j, 0, pl.ds(l * 16, 16)]
                vd = didx[j, 0, pl.ds(l * 16, 16)]
                plsc.addupdate_scatter(
                    hist_s,
                    [lax.shift_right_logical(vs, 7), lax.bitwise_and(vs, 127)],
                    ones)
                plsc.addupdate_scatter(
                    hist_d,
                    [lax.shift_right_logical(vd, 7), lax.bitwise_and(vd, 127)],
                    ones)
                return 0

            lax.fori_loop(0, CH // 16, vec, 0)
            return 0

        lax.fori_loop(0, K, chunk, 0)
        pltpu.sync_copy(hist_s, acc_s.at[iota_v], add=True)
        pltpu.sync_copy(hist_d, acc_d.at[iota_v], add=True)
        plsc.subcore_barrier()

        @pl.when(sid == 0)
        def _():
            pltpu.sync_copy(acc_s, out_hbm.at[cid, 0])
            pltpu.sync_copy(acc_d, out_hbm.at[cid, 1])

    return k(src_ch, dst_ch, iota_hbm, zeros_hbm)


def _sc_batch_gather(table, bi_ch):
    @functools.partial(
        pl.kernel,
        mesh=_sc_mesh(),
        out_type=jax.ShapeDtypeStruct((BATCH, HID), jnp.float32),
        scratch_types=[
            pltpu.VMEM((1, CH), jnp.int32),
            pltpu.VMEM((CH, HID), jnp.float32),
            pltpu.SemaphoreType.DMA,
        ],
    )
    def k(t_h, bi_h, out_h, idxv, rows, sem):
        cid = lax.axis_index("c")
        sid = lax.axis_index("s")
        wid = cid * NS + sid
        pltpu.sync_copy(bi_h.at[wid], idxv)
        pltpu.async_copy(t_h.at[idxv.at[0]], rows, sem).wait()
        pltpu.sync_copy(rows, out_h.at[pl.ds(wid * CH, CH)])

    return k(table, bi_ch)


def _tc_scale(acc2, cnt2):
    def body(a_ref, c_ref, o_ref):
        a = a_ref[0] + a_ref[1]
        c = c_ref[0] + c_ref[1]
        o_ref[...] = a / jnp.maximum(c, 1.0)

    return pl.pallas_call(
        body,
        out_shape=jax.ShapeDtypeStruct((SEG_PAD, HID), jnp.float32),
    )(acc2, cnt2)


def _tc_matmul(acc2, cnt2, W, b):
    def body(a_ref, c_ref, w_ref, b_ref, o_ref):
        a = a_ref[0] + a_ref[1]
        c = c_ref[0] + c_ref[1]
        x = a / jnp.maximum(c, 1.0)
        y = jnp.dot(x, w_ref[...], preferred_element_type=jnp.float32)
        o_ref[...] = jnp.maximum(y + b_ref[...][None, :], 0.0)

    return pl.pallas_call(
        body,
        out_shape=jax.ShapeDtypeStruct((SEG_PAD, HID), jnp.float32),
    )(acc2, cnt2, W, b)


def _tc_head(bt_rows, protos, Wc, bc):
    BB = 512
    NB = BATCH // BB

    def body(bt_ref, p_ref, wc_ref, bc_ref, o_ref):
        bt = bt_ref[...]
        p = p_ref[...]
        pf = p.reshape(N_SE * N_PROTO, HID)
        sT = lax.dot_general(
            pf, bt, (((1,), (1,)), ((), ())),
            preferred_element_type=jnp.float32)
        s3 = sT.reshape(N_SE, N_PROTO, BB)
        m = jnp.max(s3, axis=1, keepdims=True)
        e = jnp.exp(s3 - m)
        attn = e / jnp.sum(e, axis=1, keepdims=True)
        wc = wc_ref[...]
        q = jnp.sum(p * wc[:, None, :], axis=-1)
        contrib = jnp.sum(attn * q[:, :, None], axis=1)
        baseT = lax.dot_general(
            wc, bt, (((1,), (1,)), ((), ())),
            preferred_element_type=jnp.float32)
        logitsT = baseT + contrib + bc_ref[...][:, None]
        o_ref[...] = jnp.transpose(1.0 / (1.0 + jnp.exp(-logitsT)))

    return pl.pallas_call(
        body,
        grid=(NB,),
        in_specs=[
            pl.BlockSpec((BB, HID), lambda i: (i, 0)),
            pl.BlockSpec((N_SE, N_PROTO, HID), lambda i: (0, 0, 0)),
            pl.BlockSpec((N_SE, HID), lambda i: (0, 0)),
            pl.BlockSpec((N_SE,), lambda i: (0,)),
        ],
        out_specs=pl.BlockSpec((BB, N_SE), lambda i: (i, 0)),
        out_shape=jax.ShapeDtypeStruct((BATCH, N_SE), jnp.float32),
    )(bt_rows, protos, Wc, bc)


def kernel(structure_data, semantic_emb, hyperedge_index, batch_indices,
           emb_table, W1, b1, W2, b2, protos, Wc, bc):
    del structure_data, semantic_emb
    hi = hyperedge_index.astype(jnp.int32)
    src, dst = hi[0], hi[1]
    npad = PAD_INC - N_INC
    srcA = jnp.concatenate([src, jnp.zeros((npad,), jnp.int32)])
    dstA = jnp.concatenate([dst, jnp.full((npad,), DUMMY, jnp.int32)])
    dstB = jnp.concatenate([dst, jnp.zeros((npad,), jnp.int32)])
    srcB = jnp.concatenate([src, jnp.full((npad,), DUMMY, jnp.int32)])
    srcA = srcA.reshape(NCH, 1, CH)
    dstA = dstA.reshape(NCH, 1, CH)
    srcB = srcB.reshape(NCH, 1, CH)
    dstB = dstB.reshape(NCH, 1, CH)
    bi = batch_indices.astype(jnp.int32).reshape(NC * NS, 1, CH)

    zeros_hid = jnp.zeros((ZROWS, HID), jnp.float32)
    iota_h = jnp.arange(HROWS, dtype=jnp.int32)

    cnt = _sc_counts(srcB, dstA, iota_h, zeros_hid)
    cnt_nd = cnt[:, 0].reshape(NC, SEG_PAD, 1)
    cnt_he = cnt[:, 1].reshape(NC, SEG_PAD, 1)

    acc_he1 = _sc_segsum(emb_table, srcA, dstA, zeros_hid)
    he1 = _tc_scale(acc_he1, cnt_he)
    acc_nd1 = _sc_segsum(he1, dstB, srcB, zeros_hid)
    x2 = _tc_matmul(acc_nd1, cnt_nd, W1, b1)
    acc_he2 = _sc_segsum(x2, srcA, dstA, zeros_hid)
    he2 = _tc_scale(acc_he2, cnt_he)
    acc_nd2 = _sc_segsum(he2, dstB, srcB, zeros_hid)
    x3 = _tc_matmul(acc_nd2, cnt_nd, W2, b2)
    acc_t = _sc_segsum(x3, srcA, dstA, zeros_hid)
    treatment = _tc_scale(acc_t, cnt_he)

    bt_rows = _sc_batch_gather(treatment, bi)
    return _tc_head(bt_rows, protos, Wc, bc)


# speedup vs baseline: 4.1868x; 1.6397x over previous
import functools

import jax
import jax.numpy as jnp
from jax import lax
from jax.experimental import pallas as pl
from jax.experimental.pallas import tpu as pltpu
from jax.experimental.pallas import tpu_sc as plsc

N_DRUGS = 10000
N_SEG = 10000
HID = 128
N_SE = 32
N_PROTO = 20
N_INC = 320000
BATCH = 4096

NC = 2
NS = 16
CH = 16
GRP = 16
CPT = 1280
NG = CPT // GRP
NTCH = NS * CPT
PAD_INC = NTCH * CH
DUMMY = N_SEG
SEG_PAD = 10112
HALF = SEG_PAD // 2
ACC_R = 5064
DUMY_L = HALF
TBL_R = N_SEG
ZROWS = 632
HROWS = 80

KC = 80
NCHC = KC * NC * NS


def _sc_mesh():
    return plsc.VectorSubcoreMesh(core_axis_name="c", subcore_axis_name="s")


def _sc_segsum(table, cidx_all, zeros_hbm):
    @functools.partial(
        pl.kernel,
        mesh=_sc_mesh(),
        compiler_params=pltpu.CompilerParams(needs_layout_passes=False),
        out_type=jax.ShapeDtypeStruct((NC, HALF, HID), jnp.float32),
        scratch_types=[
            pltpu.VMEM((48, HID), jnp.float32),
            pltpu.VMEM_SHARED((TBL_R, HID), jnp.float32),
            pltpu.VMEM_SHARED((ACC_R, HID), jnp.float32),
            pltpu.SemaphoreType.DMA,
            pltpu.SemaphoreType.DMA,
            pltpu.SemaphoreType.DMA,
            pltpu.SemaphoreType.DMA,
            pltpu.SemaphoreType.DMA,
        ],
    )
    def k(table_hbm, cidx_hbm, zero_hbm, out_hbm,
          wsp, tbl_s, acc,
          gsem0, gsem1, asem0, asem1, isem):
        cid = lax.axis_index("c")
        sid = lax.axis_index("s")
        tgrp = sid * NG
        buf0 = wsp.at[pl.ds(0, CH)]
        buf1 = wsp.at[pl.ds(CH, CH)]
        bufs = (buf0, buf1)
        gsems = (gsem0, gsem1)

        def idx_vec(slot, c, w):
            f = 2 * c + w
            row = 32 + 8 * slot + f // 8
            col = 16 * (f % 8)
            return plsc.bitcast(wsp[row, pl.ds(col, 16)], jnp.int32)

        @pl.when(sid < NS - 1)
        def _():
            pltpu.sync_copy(table_hbm.at[pl.ds(sid * ZROWS, ZROWS)],
                            tbl_s.at[pl.ds(sid * ZROWS, ZROWS)])

        @pl.when(sid == NS - 1)
        def _():
            pltpu.sync_copy(table_hbm.at[pl.ds((NS - 1) * ZROWS,
                                               TBL_R - (NS - 1) * ZROWS)],
                            tbl_s.at[pl.ds((NS - 1) * ZROWS,
                                           TBL_R - (NS - 1) * ZROWS)])

        @pl.when(sid < NS - 1)
        def _():
            pltpu.sync_copy(zero_hbm.at[pl.ds(0, 320)],
                            acc.at[pl.ds(sid * 320, 320)])

        @pl.when(sid == NS - 1)
        def _():
            pltpu.sync_copy(zero_hbm.at[pl.ds(0, HALF - 320 * (NS - 1))],
                            acc.at[pl.ds(320 * (NS - 1),
                                         HALF - 320 * (NS - 1))])

        pltpu.sync_copy(cidx_hbm.at[cid, tgrp], wsp.at[pl.ds(32, 8)])
        plsc.subcore_barrier()

        def gstart(slot, c, b):
            pltpu.async_copy(tbl_s.at[idx_vec(slot, c, 0)],
                             bufs[b], gsems[b])

        def gwait(slot, c, b):
            pltpu.make_async_copy(tbl_s.at[idx_vec(slot, c, 0)],
                                  bufs[b], gsems[b]).wait()

        gstart(0, 0, 0)
        gstart(0, 1, 1)

        def body(g, _):
            p = jnp.bitwise_and(g, 1)
            pn = 1 - p
            gnext = jnp.minimum(g + 1, NG - 1)
            ipf = pltpu.async_copy(
                cidx_hbm.at[cid, tgrp + gnext],
                wsp.at[pl.ds(32 + 8 * pn, 8)], isem)
            for c in range(0, GRP, 2):
                gwait(p, c, 0)
                a0 = pltpu.async_copy(buf0, acc.at[idx_vec(p, c, 1)],
                                      asem0, add=True)
                gwait(p, c + 1, 1)
                a1 = pltpu.async_copy(buf1, acc.at[idx_vec(p, c + 1, 1)],
                                      asem1, add=True)
                if c + 2 < GRP:
                    a0.wait()
                    gstart(p, c + 2, 0)
                    a1.wait()
                    gstart(p, c + 3, 1)
                else:
                    ipf.wait()
                    a0.wait()
                    gstart(pn, 0, 0)
                    a1.wait()
                    gstart(pn, 1, 1)
            return 0

        lax.fori_loop(0, NG, body, 0)
        gwait(0, 0, 0)
        gwait(0, 1, 1)
        plsc.subcore_barrier()

        @pl.when(sid < NS - 1)
        def _():
            pltpu.sync_copy(acc.at[pl.ds(sid * 320, 320)],
                            out_hbm.at[cid, pl.ds(sid * 320, 320)])

        @pl.when(sid == NS - 1)
        def _():
            pltpu.sync_copy(
                acc.at[pl.ds(320 * (NS - 1), HALF - 320 * (NS - 1))],
                out_hbm.at[cid, pl.ds(320 * (NS - 1),
                                      HALF - 320 * (NS - 1))])

    return k(table, cidx_all, zeros_hbm)


def _sc_counts(src_ch, dst_ch, iota_hbm, zeros_hbm):
    @functools.partial(
        pl.kernel,
        mesh=_sc_mesh(),
        compiler_params=pltpu.CompilerParams(needs_layout_passes=False),
        out_type=jax.ShapeDtypeStruct((NC, 2, HROWS, 128), jnp.float32),
        scratch_types=[
            pltpu.VMEM((KC, 1, 128), jnp.int32),
            pltpu.VMEM((KC, 1, 128), jnp.int32),
            pltpu.VMEM((HROWS,), jnp.int32),
            pltpu.VMEM((HROWS, 128), jnp.float32),
            pltpu.VMEM((HROWS, 128), jnp.float32),
            pltpu.VMEM_SHARED((HROWS, 128), jnp.float32),
            pltpu.VMEM_SHARED((HROWS, 128), jnp.float32),
        ],
    )
    def k(sidx_hbm, didx_hbm, iota_h, zero_h, out_hbm,
          sidx, didx, iota_v, hist_s, hist_d, acc_s, acc_d):
        cid = lax.axis_index("c")
        sid = lax.axis_index("s")
        tbase = (cid * NS + sid) * KC
        pltpu.sync_copy(sidx_hbm.at[pl.ds(tbase, KC)], sidx)
        pltpu.sync_copy(didx_hbm.at[pl.ds(tbase, KC)], didx)
        pltpu.sync_copy(iota_h, iota_v)
        pltpu.sync_copy(zero_h.at[pl.ds(0, HROWS)], hist_s)
        pltpu.sync_copy(zero_h.at[pl.ds(0, HROWS)], hist_d)

        @pl.when(sid == 0)
        def _():
            pltpu.sync_copy(zero_h.at[pl.ds(0, HROWS)], acc_s)
            pltpu.sync_copy(zero_h.at[pl.ds(0, HROWS)], acc_d)

        plsc.subcore_barrier()
        ones = jnp.ones((16,), jnp.float32)

        def chunk(j, _):
            def vec(l, _2):
                vs = sidx[j, 0, pl.ds(l * 16, 16)]
                vd = didx[j, 0, pl.ds(l * 16, 16)]
                plsc.addupdate_scatter(
                    hist_s,
                    [lax.shift_right_logical(vs, 7), lax.bitwise_and(vs, 127)],
                    ones)
                plsc.addupdate_scatter(
                    hist_d,
                    [lax.shift_right_logical(vd, 7), lax.bitwise_and(vd, 127)],
                    ones)
                return 0

            lax.fori_loop(0, 8, vec, 0)
            return 0

        lax.fori_loop(0, KC, chunk, 0)
        pltpu.sync_copy(hist_s, acc_s.at[iota_v], add=True)
        pltpu.sync_copy(hist_d, acc_d.at[iota_v], add=True)
        plsc.subcore_barrier()

        @pl.when(sid == 0)
        def _():
            pltpu.sync_copy(acc_s, out_hbm.at[cid, 0])
            pltpu.sync_copy(acc_d, out_hbm.at[cid, 1])

    return k(src_ch, dst_ch, iota_hbm, zeros_hbm)


def _sc_batch_gather(table, bi_ch):
    @functools.partial(
        pl.kernel,
        mesh=_sc_mesh(),
        out_type=jax.ShapeDtypeStruct((BATCH, HID), jnp.float32),
        scratch_types=[
            pltpu.VMEM((1, 128), jnp.int32),
            pltpu.VMEM((128, HID), jnp.float32),
            pltpu.SemaphoreType.DMA,
        ],
    )
    def k(t_h, bi_h, out_h, idxv, rows, sem):
        cid = lax.axis_index("c")
        sid = lax.axis_index("s")
        wid = cid * NS + sid
        pltpu.sync_copy(bi_h.at[wid], idxv)
        pltpu.async_copy(t_h.at[idxv.at[0]], rows, sem).wait()
        pltpu.sync_copy(rows, out_h.at[pl.ds(wid * 128, 128)])

    return k(table, bi_ch)


def _tc_scale(acc2, cnt2):
    def body(a_ref, c_ref, o_ref):
        a = a_ref[0]
        c = c_ref[0] + c_ref[1]
        o_ref[...] = a / jnp.maximum(c, 1.0)

    return pl.pallas_call(
        body,
        grid=(NC,),
        in_specs=[
            pl.BlockSpec((1, HALF, HID), lambda i: (i, 0, 0)),
            pl.BlockSpec((NC, HALF, 1), lambda i: (0, i, 0)),
        ],
        out_specs=pl.BlockSpec((HALF, HID), lambda i: (i, 0)),
        out_shape=jax.ShapeDtypeStruct((SEG_PAD, HID), jnp.float32),
    )(acc2, cnt2)


def _tc_matmul(acc2, cnt2, W, b):
    def body(a_ref, c_ref, w_ref, b_ref, o_ref):
        a = a_ref[0]
        c = c_ref[0] + c_ref[1]
        x = a / jnp.maximum(c, 1.0)
        y = jnp.dot(x, w_ref[...], preferred_element_type=jnp.float32)
        o_ref[...] = jnp.maximum(y + b_ref[...][None, :], 0.0)

    return pl.pallas_call(
        body,
        grid=(NC,),
        in_specs=[
            pl.BlockSpec((1, HALF, HID), lambda i: (i, 0, 0)),
            pl.BlockSpec((NC, HALF, 1), lambda i: (0, i, 0)),
            pl.BlockSpec((HID, HID), lambda i: (0, 0)),
            pl.BlockSpec((HID,), lambda i: (0,)),
        ],
        out_specs=pl.BlockSpec((HALF, HID), lambda i: (i, 0)),
        out_shape=jax.ShapeDtypeStruct((SEG_PAD, HID), jnp.float32),
    )(acc2, cnt2, W, b)


def _tc_head(bt_rows, protos, Wc, bc):
    BB = 512
    NB = BATCH // BB

    def body(bt_ref, p_ref, wc_ref, bc_ref, o_ref):
        bt = bt_ref[...]
        p = p_ref[...]
        pf = p.reshape(N_SE * N_PROTO, HID)
        sT = lax.dot_general(
            pf, bt, (((1,), (1,)), ((), ())),
            preferred_element_type=jnp.float32)
        s3 = sT.reshape(N_SE, N_PROTO, BB)
        m = jnp.max(s3, axis=1, keepdims=True)
        e = jnp.exp(s3 - m)
        attn = e / jnp.sum(e, axis=1, keepdims=True)
        wc = wc_ref[...]
        q = jnp.sum(p * wc[:, None, :], axis=-1)
        contrib = jnp.sum(attn * q[:, :, None], axis=1)
        baseT = lax.dot_general(
            wc, bt, (((1,), (1,)), ((), ())),
            preferred_element_type=jnp.float32)
        logitsT = baseT + contrib + bc_ref[...][:, None]
        o_ref[...] = jnp.transpose(1.0 / (1.0 + jnp.exp(-logitsT)))

    return pl.pallas_call(
        body,
        grid=(NB,),
        in_specs=[
            pl.BlockSpec((BB, HID), lambda i: (i, 0)),
            pl.BlockSpec((N_SE, N_PROTO, HID), lambda i: (0, 0, 0)),
            pl.BlockSpec((N_SE, HID), lambda i: (0, 0)),
            pl.BlockSpec((N_SE,), lambda i: (0,)),
        ],
        out_specs=pl.BlockSpec((BB, N_SE), lambda i: (i, 0)),
        out_shape=jax.ShapeDtypeStruct((BATCH, N_SE), jnp.float32),
    )(bt_rows, protos, Wc, bc)


def kernel(structure_data, semantic_emb, hyperedge_index, batch_indices,
           emb_table, W1, b1, W2, b2, protos, Wc, bc):
    del structure_data, semantic_emb
    hi = hyperedge_index.astype(jnp.int32)
    src, dst = hi[0], hi[1]
    npad = PAD_INC - N_INC
    zpad = jnp.zeros((npad,), jnp.int32)
    dpad = jnp.full((npad,), DUMMY, jnp.int32)
    src_g = jnp.concatenate([src, zpad])
    dst_g = jnp.concatenate([dst, zpad])
    src_s = jnp.concatenate([src, dpad])
    dst_s = jnp.concatenate([dst, dpad])

    def combined(gth, sct):
        lo = jnp.where(sct < HALF, sct, DUMY_L)
        hia = jnp.where((sct >= HALF) & (sct < N_SEG), sct - HALF, DUMY_L)
        s2 = jnp.stack([lo, hia]).reshape(NC, NTCH, 1, 1, CH)
        g2 = jnp.broadcast_to(gth.reshape(1, NTCH, 1, 1, CH),
                              (NC, NTCH, 1, 1, CH))
        c4 = jnp.concatenate([g2, s2], axis=2)
        flat = c4.reshape(NC, NS * NG, GRP * 2 * CH)
        f32v = lax.bitcast_convert_type(flat, jnp.float32)
        arr = f32v.reshape(NC, NS * NG, 4, 128)
        return jnp.pad(arr, ((0, 0), (0, 0), (0, 4), (0, 0)))

    c_by_dst = combined(src_g, dst_s)
    c_by_src = combined(dst_g, src_s)
    src_c = src_s.reshape(NCHC, 1, 128)
    dst_c = dst_s.reshape(NCHC, 1, 128)
    bi = batch_indices.astype(jnp.int32).reshape(NC * NS, 1, 128)

    zeros_hid = jnp.zeros((ZROWS, HID), jnp.float32)
    iota_h = jnp.arange(HROWS, dtype=jnp.int32)

    cnt = _sc_counts(src_c, dst_c, iota_h, zeros_hid)
    cnt_nd = cnt[:, 0].reshape(NC, HROWS * 128, 1)[:, :SEG_PAD]
    cnt_he = cnt[:, 1].reshape(NC, HROWS * 128, 1)[:, :SEG_PAD]

    acc_he1 = _sc_segsum(emb_table, c_by_dst, zeros_hid)
    he1 = _tc_scale(acc_he1, cnt_he)
    acc_nd1 = _sc_segsum(he1, c_by_src, zeros_hid)
    x2 = _tc_matmul(acc_nd1, cnt_nd, W1, b1)
    acc_he2 = _sc_segsum(x2, c_by_dst, zeros_hid)
    he2 = _tc_scale(acc_he2, cnt_he)
    acc_nd2 = _sc_segsum(he2, c_by_src, zeros_hid)
    x3 = _tc_matmul(acc_nd2, cnt_nd, W2, b2)
    acc_t = _sc_segsum(x3, c_by_dst, zeros_hid)
    treatment = _tc_scale(acc_t, cnt_he)

    bt_rows = _sc_batch_gather(treatment, bi)
    return _tc_head(bt_rows, protos, Wc, bc)
